# baseline (speedup 1.0000x reference)
import jax
import jax.numpy as jnp
from jax import lax
from jax.experimental import pallas as pl
from jax.experimental.pallas import tpu as pltpu
from jax.experimental.pallas import tpu_sc as plsc

L_DIM = 100000
EMBED_DIM = 32
R = L_DIM - 1
NC = 2
NS = 16
NW = NC * NS
LANES = 16

TCW = 128
SPAN = 25
NB = SPAN
NBUF = 4
UNIFORM_TC = 780
LAST_F = UNIFORM_TC - SPAN
TAIL0 = UNIFORM_TC * TCW
KPV = TCW // LANES

CC = 3128
CC_LAST = R - (NW - 1) * CC
CBUF = 3136


RPAD = UNIFORM_TC * TCW + 2 * TCW
TBUF = TCW


def _sc_body(wu, wv, tu, tv, out_u, out_v, out_c, biu, bou, biv, bov,
             buf_c, btu, btv, s_ui, s_vi, s_uo, s_vo, s_c):
    wid = lax.axis_index("s") * NC + lax.axis_index("c")
    iota16 = lax.iota(jnp.int32, LANES)
    perm = jnp.where(iota16 < LANES - 1, iota16 + 1, 0)
    low15 = iota16 < LANES - 1

    def rot(v):
        return v.at[perm].get(mode="promise_in_bounds", unique_indices=True)

    def elu1(y):
        return jnp.where(y > 0.0, y + 1.0, jnp.exp(y))

    cbase = wid * CC + 1

    def c_body(j, carry):
        buf_c[pl.ds(j * LANES, LANES)] = cbase + j * LANES + iota16
        return carry

    lax.fori_loop(0, CBUF // LANES, c_body, 0, unroll=4)

    @pl.when(wid < NW - 1)
    def _():
        pltpu.async_copy(buf_c.at[pl.ds(0, CC)],
                         out_c.at[pl.ds(wid * CC, CC)], s_c).wait()

    @pl.when(wid == NW - 1)
    def _():
        pltpu.async_copy(buf_c.at[pl.ds(0, CC_LAST)],
                         out_c.at[pl.ds((NW - 1) * CC, CC_LAST)], s_c).wait()

    col0 = (wid * LAST_F) // (NW - 1) * TCW

    def shift_block(src_u, src_v, la_u, la_v, dst_u, dst_v):
        def d_body(d, carry):
            ru = rot(src_u[d, pl.ds(0, LANES)])
            rv = rot(src_v[d, pl.ds(0, LANES)])
            for k in range(KPV):
                if k < KPV - 1:
                    nu = src_u[d, pl.ds((k + 1) * LANES, LANES)]
                    nv = src_v[d, pl.ds((k + 1) * LANES, LANES)]
                else:
                    nu = la_u[d, pl.ds(0, LANES)]
                    nv = la_v[d, pl.ds(0, LANES)]
                ru_b, rv_b = rot(nu), rot(nv)
                dst_u[d, pl.ds(k * LANES, LANES)] = jnp.where(low15, ru, ru_b)
                dst_v[d, pl.ds(k * LANES, LANES)] = elu1(
                    jnp.where(low15, rv, rv_b))
                ru, rv = ru_b, rv_b
            return carry

        lax.fori_loop(0, EMBED_DIM, d_body, 0)

    in_copies = {}

    def issue_in(b):
        sl = b % NBUF
        t = col0 + b * TCW
        in_copies[b] = (
            pltpu.async_copy(wu.at[:, pl.ds(t, TCW)], biu[sl], s_ui[sl]),
            pltpu.async_copy(wv.at[:, pl.ds(t, TCW)], biv[sl], s_vi[sl]),
        )

    out_copies = {}
    for b in range(NBUF):
        issue_in(b)
    cu, cv = in_copies.pop(0)
    cu.wait()
    cv.wait()

    for b in range(NB):
        sl = b % NBUF
        la = (b + 1) % NBUF
        cu, cv = in_copies.pop(b + 1)
        cu.wait()
        cv.wait()
        if b - NBUF >= 0:
            pu, pv = out_copies.pop(b - NBUF)
            pu.wait()
            pv.wait()
        shift_block(biu[sl], biv[sl], biu[la], biv[la], bou[sl], bov[sl])
        t = col0 + b * TCW
        out_copies[b] = (
            pltpu.async_copy(bou[sl], out_u.at[:, pl.ds(t, TCW)], s_uo[sl]),
            pltpu.async_copy(bov[sl], out_v.at[:, pl.ds(t, TCW)], s_vo[sl]),
        )
        if b + NBUF <= NB:
            issue_in(b + NBUF)

    for b in sorted(out_copies):
        pu, pv = out_copies[b]
        pu.wait()
        pv.wait()

    @pl.when(wid == 0)
    def _():
        pltpu.async_copy(wu.at[:, pl.ds(TAIL0, TCW)], biu[0], s_ui[0]).wait()
        pltpu.async_copy(wv.at[:, pl.ds(TAIL0, TCW)], biv[0], s_vi[0]).wait()
        pltpu.async_copy(tu, btu, s_ui[1]).wait()
        pltpu.async_copy(tv, btv, s_vi[1]).wait()
        shift_block(biu[0], biv[0], btu, btv, bou[0], bov[0])
        pltpu.async_copy(bou[0], out_u.at[:, pl.ds(TAIL0, TCW)],
                         s_uo[0]).wait()
        pltpu.async_copy(bov[0], out_v.at[:, pl.ds(TAIL0, TCW)],
                         s_vo[0]).wait()

    @pl.when(wid == 1)
    def _():
        pltpu.async_copy(tu, btu, s_ui[1]).wait()
        pltpu.async_copy(tv, btv, s_vi[1]).wait()

        def d_body(d, carry):
            ru = rot(btu[d, pl.ds(0, LANES)])
            rv = rot(btv[d, pl.ds(0, LANES)])
            for k in range(2):
                ru_b = rot(btu[d, pl.ds((k + 1) * LANES, LANES)])
                rv_b = rot(btv[d, pl.ds((k + 1) * LANES, LANES)])
                bou[0][d, pl.ds(k * LANES, LANES)] = jnp.where(low15, ru, ru_b)
                bov[0][d, pl.ds(k * LANES, LANES)] = elu1(
                    jnp.where(low15, rv, rv_b))
                ru, rv = ru_b, rv_b
            z = jnp.zeros((LANES,), jnp.float32)
            for k in range(2, KPV):
                bou[0][d, pl.ds(k * LANES, LANES)] = z
                bov[0][d, pl.ds(k * LANES, LANES)] = z
            return carry

        lax.fori_loop(0, EMBED_DIM, d_body, 0)
        pltpu.async_copy(bou[0], out_u.at[:, pl.ds(TAIL0 + TCW, TCW)],
                         s_uo[0]).wait()
        pltpu.async_copy(bov[0], out_v.at[:, pl.ds(TAIL0 + TCW, TCW)],
                         s_vo[0]).wait()


_sc_lookup = pl.kernel(
    _sc_body,
    out_type=(
        jax.ShapeDtypeStruct((EMBED_DIM, RPAD), jnp.float32),
        jax.ShapeDtypeStruct((EMBED_DIM, RPAD), jnp.float32),
        jax.ShapeDtypeStruct((R,), jnp.int32),
    ),
    mesh=plsc.VectorSubcoreMesh(core_axis_name="c", subcore_axis_name="s",
                                num_cores=NC, num_subcores=NS),
    compiler_params=pltpu.CompilerParams(use_tc_tiling_on_sc=True),
    scratch_types=[
        [pltpu.VMEM((EMBED_DIM, TCW), jnp.float32)] * NBUF,
        [pltpu.VMEM((EMBED_DIM, TCW), jnp.float32)] * NBUF,
        [pltpu.VMEM((EMBED_DIM, TCW), jnp.float32)] * NBUF,
        [pltpu.VMEM((EMBED_DIM, TCW), jnp.float32)] * NBUF,
        pltpu.VMEM((CBUF,), jnp.int32),
        pltpu.VMEM((EMBED_DIM, TBUF), jnp.float32),
        pltpu.VMEM((EMBED_DIM, TBUF), jnp.float32),
        [pltpu.SemaphoreType.DMA] * NBUF,
        [pltpu.SemaphoreType.DMA] * NBUF,
        [pltpu.SemaphoreType.DMA] * NBUF,
        [pltpu.SemaphoreType.DMA] * NBUF,
        pltpu.SemaphoreType.DMA,
    ],
)

def _tail_pad(wt):
    t = lax.slice(wt, (0, TAIL0 + TCW), (EMBED_DIM, L_DIM))
    return jnp.pad(t, ((0, 0), (0, TBUF - (L_DIM - TAIL0 - TCW))))


def kernel(traj, traj_len, W_u, W_v):
    del traj, traj_len
    wut = jnp.swapaxes(W_u, 0, 1)
    wvt = jnp.swapaxes(W_v, 0, 1)
    scu, scv, candidates = _sc_lookup(wut, wvt, _tail_pad(wut),
                                      _tail_pad(wvt))
    out_u_t = lax.slice(scu, (0, 0), (EMBED_DIM, R))
    out_v_t = lax.slice(scv, (0, 0), (EMBED_DIM, R))
    return (jnp.swapaxes(out_u_t, 0, 1), jnp.swapaxes(out_v_t, 0, 1),
            candidates)
